# baseline (speedup 1.0000x reference)
import jax
import jax.numpy as jnp
from jax import lax
from jax.experimental import pallas as pl
from jax.experimental.pallas import tpu as pltpu
from jax.experimental.pallas import tpu_sc as plsc

B, L, D = 4, 8192, 1024
N_TOK = B * L
NC, NS = 2, 16
NW = NC * NS
TOK_PER_W = N_TOK // NW
N_GRP16 = TOK_PER_W // 16
REP = 32


def _sc_body(table_hbm, idx_hbm, out_hbm, idx_v, l0_v, l1_v, t0_v, t1_v,
             rep0, rep1, sem):
    wid = lax.axis_index("s") * NC + lax.axis_index("c")
    base = wid * TOK_PER_W
    idx_cp = pltpu.async_copy(idx_hbm.at[pl.ds(base, TOK_PER_W)], idx_v, sem)
    pltpu.sync_copy(table_hbm.at[0], t0_v)
    pltpu.sync_copy(table_hbm.at[1], t1_v)

    def fill(g, c):
        x0 = t0_v[pl.ds(g * 16, 16)]
        x1 = t1_v[pl.ds(g * 16, 16)]
        for j in range(REP):
            rep0[j, pl.ds(g * 16, 16)] = x0
            rep1[j, pl.ds(g * 16, 16)] = x1
        return c

    lax.fori_loop(0, D // 16, fill, 0, unroll=2)
    idx_cp.wait()

    lane = lax.iota(jnp.int32, 16)

    def compact(i, carry):
        c0, c1 = carry
        pos = lane + (base + i * 16)
        idvi = idx_v[pl.ds(i * 16, 16)]
        m1 = idvi != 0
        m0 = jnp.logical_not(m1)
        p = jnp.minimum(idvi, 1)
        for k in (1, 2, 4, 8):
            mk = lax.shift_right_arithmetic(lane - k, 31)
            g = p.at[jnp.maximum(lane - k, 0)].get(mode="promise_in_bounds")
            p = p + (g & ~mk)
        dest1 = jnp.maximum(c1 + p - 1, 0)
        dest0 = jnp.maximum(c0 + lane - p, 0)
        plsc.store_scatter(l0_v, [dest0], pos, mask=m0)
        plsc.store_scatter(l1_v, [dest1], pos, mask=m1)
        cnt1 = p.at[jnp.full((16,), 15, jnp.int32)].get(
            mode="promise_in_bounds"
        )
        return c0 + (16 - cnt1), c1 + cnt1

    zero16 = jnp.zeros((16,), jnp.int32)
    c0f, c1f = lax.fori_loop(0, N_GRP16, compact, (zero16, zero16))
    n0 = c0f[0]
    n1 = c1f[0]

    def scatter_value(list_v, rep, nv):
        @pl.when(nv > 0)
        def _go():
            pad = (-nv) % REP

            @pl.when(pad > 0)
            def _pad():
                w16 = list_v[pl.ds(((nv - 1) // 16) * 16, 16)]
                lastv = w16.at[
                    jnp.broadcast_to((nv - 1) % 16, (16,)).astype(jnp.int32)
                ].get(mode="promise_in_bounds")
                plsc.store_scatter(
                    list_v, [nv + lane], lastv, mask=lane < pad
                )
                plsc.store_scatter(
                    list_v, [nv + 16 + lane], lastv, mask=lane < pad - 16
                )

            def win(w, c):
                pltpu.async_copy(
                    rep, out_hbm.at[list_v.at[pl.ds(w * REP, REP)]], sem
                )
                return c

            lax.fori_loop(0, (nv + REP - 1) // REP, win, 0)

        return jnp.where(nv > 0, (nv + REP - 1) // REP, 0)

    ns0 = scatter_value(l0_v, rep0, n0)
    ns1 = scatter_value(l1_v, rep1, n1)

    def drain(i, c):
        pltpu.make_async_copy(
            rep0, out_hbm.at[l0_v.at[pl.ds(0, REP)]], sem
        ).wait()
        return c

    lax.fori_loop(0, ns0 + ns1, drain, 0)


@jax.jit
def _lookup(ids_flat, table):
    mesh = plsc.VectorSubcoreMesh(core_axis_name="c", subcore_axis_name="s")
    run = pl.kernel(
        _sc_body,
        out_type=jax.ShapeDtypeStruct((N_TOK, D), jnp.float32),
        mesh=mesh,
        compiler_params=pltpu.CompilerParams(needs_layout_passes=False),
        scratch_types=[
            pltpu.VMEM((TOK_PER_W,), jnp.int32),
            pltpu.VMEM((TOK_PER_W + 16,), jnp.int32),
            pltpu.VMEM((TOK_PER_W + 16,), jnp.int32),
            pltpu.VMEM((D,), jnp.float32),
            pltpu.VMEM((D,), jnp.float32),
            pltpu.VMEM((REP, D), jnp.float32),
            pltpu.VMEM((REP, D), jnp.float32),
            pltpu.SemaphoreType.DMA,
        ],
    )
    return run(table, ids_flat)


def kernel(token_type_ids, table):
    ids_flat = token_type_ids.reshape(-1).astype(jnp.int32)
    out = _lookup(ids_flat, table)
    return out.reshape(token_type_ids.shape + (D,))


# speedup vs baseline: 1.0591x; 1.0591x over previous
import jax
import jax.numpy as jnp
from jax import lax
from jax.experimental import pallas as pl
from jax.experimental.pallas import tpu as pltpu
from jax.experimental.pallas import tpu_sc as plsc

B, L, D = 4, 8192, 1024
N_TOK = B * L
NC, NS = 2, 16
NW = NC * NS
TOK_PER_W = N_TOK // NW
N_GRP16 = TOK_PER_W // 16
REP = 16


def _sc_body(table_hbm, idx_hbm, out_hbm, idx_v, l0_v, l1_v, t0_v, t1_v,
             rep0, rep1, sem):
    wid = lax.axis_index("s") * NC + lax.axis_index("c")
    base = wid * TOK_PER_W
    idx_cp = pltpu.async_copy(idx_hbm.at[pl.ds(base, TOK_PER_W)], idx_v, sem)
    pltpu.sync_copy(table_hbm.at[0], t0_v)
    pltpu.sync_copy(table_hbm.at[1], t1_v)

    def fill(g, c):
        x0 = t0_v[pl.ds(g * 16, 16)]
        x1 = t1_v[pl.ds(g * 16, 16)]
        for j in range(REP):
            rep0[j, pl.ds(g * 16, 16)] = x0
            rep1[j, pl.ds(g * 16, 16)] = x1
        return c

    lax.fori_loop(0, D // 16, fill, 0, unroll=2)
    idx_cp.wait()

    lane = lax.iota(jnp.int32, 16)

    def compact(i, carry):
        c0, c1 = carry
        pos = lane + (base + i * 16)
        idvi = idx_v[pl.ds(i * 16, 16)]
        m1 = idvi != 0
        m0 = jnp.logical_not(m1)
        p = jnp.minimum(idvi, 1)
        for k in (1, 2, 4, 8):
            mk = lax.shift_right_arithmetic(lane - k, 31)
            g = p.at[jnp.maximum(lane - k, 0)].get(mode="promise_in_bounds")
            p = p + (g & ~mk)
        dest1 = jnp.maximum(c1 + p - 1, 0)
        dest0 = jnp.maximum(c0 + lane - p, 0)
        plsc.store_scatter(l0_v, [dest0], pos, mask=m0)
        plsc.store_scatter(l1_v, [dest1], pos, mask=m1)
        cnt1 = p.at[jnp.full((16,), 15, jnp.int32)].get(
            mode="promise_in_bounds"
        )
        return c0 + (16 - cnt1), c1 + cnt1

    zero16 = jnp.zeros((16,), jnp.int32)
    c0f, c1f = lax.fori_loop(0, N_GRP16, compact, (zero16, zero16))
    n0 = c0f[0]
    n1 = c1f[0]

    def scatter_value(list_v, rep, nv):
        @pl.when(nv > 0)
        def _go():
            pad = (-nv) % REP

            @pl.when(pad > 0)
            def _pad():
                w16 = list_v[pl.ds(((nv - 1) // 16) * 16, 16)]
                lastv = w16.at[
                    jnp.broadcast_to((nv - 1) % 16, (16,)).astype(jnp.int32)
                ].get(mode="promise_in_bounds")
                plsc.store_scatter(
                    list_v, [nv + lane], lastv, mask=lane < pad
                )

            def win(w, c):
                pltpu.async_copy(
                    rep, out_hbm.at[list_v.at[pl.ds(w * REP, REP)]], sem
                )
                return c

            lax.fori_loop(0, (nv + REP - 1) // REP, win, 0)

        return jnp.where(nv > 0, (nv + REP - 1) // REP, 0)

    ns0 = scatter_value(l0_v, rep0, n0)
    ns1 = scatter_value(l1_v, rep1, n1)

    def drain(i, c):
        pltpu.make_async_copy(
            rep0, out_hbm.at[l0_v.at[pl.ds(0, REP)]], sem
        ).wait()
        return c

    lax.fori_loop(0, ns0 + ns1, drain, 0)


@jax.jit
def _lookup(ids_flat, table):
    mesh = plsc.VectorSubcoreMesh(core_axis_name="c", subcore_axis_name="s")
    run = pl.kernel(
        _sc_body,
        out_type=jax.ShapeDtypeStruct((N_TOK, D), jnp.float32),
        mesh=mesh,
        compiler_params=pltpu.CompilerParams(needs_layout_passes=False),
        scratch_types=[
            pltpu.VMEM((TOK_PER_W,), jnp.int32),
            pltpu.VMEM((TOK_PER_W + 16,), jnp.int32),
            pltpu.VMEM((TOK_PER_W + 16,), jnp.int32),
            pltpu.VMEM((D,), jnp.float32),
            pltpu.VMEM((D,), jnp.float32),
            pltpu.VMEM((REP, D), jnp.float32),
            pltpu.VMEM((REP, D), jnp.float32),
            pltpu.SemaphoreType.DMA,
        ],
    )
    return run(table, ids_flat)


def kernel(token_type_ids, table):
    ids_flat = token_type_ids.reshape(-1).astype(jnp.int32)
    out = _lookup(ids_flat, table)
    return out.reshape(token_type_ids.shape + (D,))
